# baseline (speedup 1.0000x reference)
import functools

import jax
import jax.numpy as jnp
from jax import lax
from jax.experimental import pallas as pl
from jax.experimental.pallas import tpu as pltpu
from jax.experimental.pallas import tpu_sc as plsc

NC = 2
NS = 16
LANES = 16
NW = NC * NS
BATCH = 16384
HIDDEN = 64
WIDE = 2 * HIDDEN
B_PER_W = BATCH // NW
CHUNK = 128
NCHUNK = B_PER_W // CHUNK
RING = 2
GGROUP = CHUNK // LANES
EPS_SQ = 1e-16


def _rsqrt(x):
    i = plsc.bitcast(x, jnp.int32)
    y = plsc.bitcast(jnp.int32(0x5F3759DF) - (i >> 1), jnp.float32)
    xh = x * jnp.float32(0.5)
    for _ in range(3):
        y = y * (jnp.float32(1.5) - xh * y * y)
    return y


_mesh = plsc.VectorSubcoreMesh(core_axis_name="c", subcore_axis_name="s")


@functools.partial(
    pl.kernel,
    out_type=jax.ShapeDtypeStruct((NW, NCHUNK, CHUNK), jnp.float32),
    mesh=_mesh,
    scratch_types=[
        pltpu.VMEM((NCHUNK, CHUNK), jnp.int32),
        pltpu.VMEM((NCHUNK, CHUNK), jnp.int32),
        pltpu.VMEM((NCHUNK, CHUNK), jnp.int32),
        pltpu.VMEM((NCHUNK, CHUNK), jnp.int32),
        pltpu.VMEM((RING, CHUNK, WIDE), jnp.float32),
        pltpu.VMEM((RING, CHUNK, WIDE), jnp.float32),
        pltpu.VMEM((NCHUNK, CHUNK), jnp.float32),
        [pltpu.SemaphoreType.DMA] * RING,
    ],
    compiler_params=pltpu.CompilerParams(
        needs_layout_passes=False, use_tc_tiling_on_sc=True),
)
def _cosine_kernel(row1_hbm, row2_hbm, off1_hbm, off2_hbm, table_hbm,
                   out_hbm, row1_v, row2_v, off1_v, off2_v,
                   buf1_v, buf2_v, out_v, sems):
    wid = lax.axis_index("s") * NC + lax.axis_index("c")

    pltpu.sync_copy(row1_hbm.at[wid], row1_v)
    pltpu.sync_copy(row2_hbm.at[wid], row2_v)
    pltpu.sync_copy(off1_hbm.at[wid], off1_v)
    pltpu.sync_copy(off2_hbm.at[wid], off2_v)

    def fire_chunk(c, slot):
        pltpu.async_copy(
            table_hbm.at[row1_v.at[c]], buf1_v.at[slot], sems[slot])
        pltpu.async_copy(
            table_hbm.at[row2_v.at[c]], buf2_v.at[slot], sems[slot])

    def drain_chunk(slot):
        pltpu.make_async_copy(
            table_hbm.at[pl.ds(0, CHUNK)], buf1_v.at[slot], sems[slot]
        ).wait()
        pltpu.make_async_copy(
            table_hbm.at[pl.ds(0, CHUNK)], buf2_v.at[slot], sems[slot]
        ).wait()

    iota = lax.iota(jnp.int32, LANES)
    zeros = jnp.zeros((LANES,), jnp.float32)

    def compute_chunk(c, slot):
        def group_body(g, carry):
            rowpos = iota + g * LANES
            off1 = off1_v[c, pl.ds(g * LANES, LANES)]
            off2 = off2_v[c, pl.ds(g * LANES, LANES)]
            dot = zeros
            s1 = zeros
            s2 = zeros
            for d in range(HIDDEN):
                v1 = plsc.load_gather(buf1_v.at[slot], [rowpos, off1 + d])
                v2 = plsc.load_gather(buf2_v.at[slot], [rowpos, off2 + d])
                dot = dot + v1 * v2
                s1 = s1 + v1 * v1
                s2 = s2 + v2 * v2
            denom_sq = jnp.maximum(s1, EPS_SQ) * jnp.maximum(s2, EPS_SQ)
            out_v[c, pl.ds(g * LANES, LANES)] = dot * _rsqrt(denom_sq)
            return carry

        lax.fori_loop(0, GGROUP, group_body, 0)

    for r in range(RING):
        fire_chunk(r, r)

    for c in range(NCHUNK):
        slot = c % RING
        drain_chunk(slot)
        compute_chunk(c, slot)
        if c + RING < NCHUNK:
            fire_chunk(c + RING, slot)

    pltpu.sync_copy(out_v, out_hbm.at[wid])


def kernel(first_item, second_item, item_embedding):
    first = first_item.astype(jnp.int32)
    second = second_item.astype(jnp.int32)
    shape3 = (NW, NCHUNK, CHUNK)
    row1 = (first >> 1).reshape(shape3)
    row2 = (second >> 1).reshape(shape3)
    off1 = ((first & 1) * HIDDEN).reshape(shape3)
    off2 = ((second & 1) * HIDDEN).reshape(shape3)
    table2 = item_embedding.reshape(500000, WIDE)
    out = _cosine_kernel(row1, row2, off1, off2, table2)
    return out.reshape(BATCH)
